# Initial kernel scaffold; baseline (speedup 1.0000x reference)
import jax
import jax.numpy as jnp
from jax.experimental import pallas as pl


def kernel(user_emb, item_emb, edge_values, W_gc, b_gc, W_bi, b_bi, edge_index, users, items):
    raise NotImplementedError("write your pallas kernel here")


# speedup vs baseline: 5.0365x; 5.0365x over previous
import functools

import jax
import jax.numpy as jnp
from jax import lax
from jax.experimental import pallas as pl
from jax.experimental.pallas import tpu as pltpu
from jax.experimental.pallas import tpu_sc as plsc

N_USER = 25000
N_ITEM = 25000
NNODE = N_USER + N_ITEM
D = 64
DH = 32
L = 3
E = 800000
B = 4096

NC, NS = 2, 16
K = 512
NBLK = 98
NSUB = K // 128
ET = NBLK * K
EP = NS * ET
NPAD = 50176
RPT = NPAD // NS
RB = 512
GRID = NPAD // RB

_mesh = plsc.VectorSubcoreMesh(core_axis_name="c", subcore_axis_name="s")
_sc_params = pltpu.CompilerParams(use_tc_tiling_on_sc=False)


@functools.partial(
    pl.kernel,
    out_type=jax.ShapeDtypeStruct((NC, NPAD, DH), jnp.float32),
    mesh=_mesh,
    scratch_types=[
        pltpu.MemorySpace.VMEM_SHARED((NPAD, DH), jnp.float32),
        pltpu.MemorySpace.VMEM((NSUB, 128), jnp.int32),
        pltpu.MemorySpace.VMEM((NSUB, 128), jnp.int32),
        pltpu.MemorySpace.VMEM((K,), jnp.float32),
        pltpu.MemorySpace.VMEM((K, DH), jnp.float32),
        pltpu.SemaphoreType.DMA,
        pltpu.SemaphoreType.DMA,
    ],
    compiler_params=_sc_params,
)
def _spmm(tbl_hbm, col_hbm, row_hbm, val_hbm, out_hbm,
          acc, colb, rowb, valb, gath, gsem, ssem):
    c = lax.axis_index("c")
    t = lax.axis_index("s")

    def zrow(i, _):
        gath[i, pl.ds(0, 16)] = jnp.zeros((16,), jnp.float32)
        gath[i, pl.ds(16, 16)] = jnp.zeros((16,), jnp.float32)
        return 0
    lax.fori_loop(0, K, zrow, 0)
    for off in range(0, RPT, K):
        sz = min(K, RPT - off)
        pltpu.sync_copy(gath.at[pl.ds(0, sz)],
                        acc.at[pl.ds(t * RPT + off, sz)])
    plsc.subcore_barrier()

    off_c = c * NPAD

    def block_body(b, _):
        pltpu.sync_copy(col_hbm.at[t, b], colb)
        pltpu.sync_copy(row_hbm.at[t, b], rowb)
        pltpu.sync_copy(val_hbm.at[t, b], valb)

        def obody(i, _):
            for j in range(NSUB):
                colb[j, pl.ds(i * 16, 16)] = colb[j, pl.ds(i * 16, 16)] + off_c
            return 0
        lax.fori_loop(0, 8, obody, 0, unroll=True)

        gd = [pltpu.async_copy(tbl_hbm.at[colb.at[j]],
                               gath.at[pl.ds(j * 128, 128)], gsem)
              for j in range(NSUB)]
        for dd in gd:
            dd.wait()

        def sbody(g, _):
            vv = valb[pl.ds(g * 16, 16)]
            base = g * 16
            for j in range(16):
                v = vv[j]
                gath[base + j, pl.ds(0, 16)] = gath[base + j, pl.ds(0, 16)] * v
                gath[base + j, pl.ds(16, 16)] = gath[base + j, pl.ds(16, 16)] * v
            return 0
        lax.fori_loop(0, K // 16, sbody, 0)

        sd = [pltpu.async_copy(gath.at[pl.ds(j * 128, 128)],
                               acc.at[rowb.at[j]], ssem, add=True)
              for j in range(NSUB)]
        for dd in sd:
            dd.wait()
        return 0

    lax.fori_loop(0, NBLK, block_body, 0)
    plsc.subcore_barrier()
    pltpu.sync_copy(acc.at[pl.ds(t * RPT, RPT)],
                    out_hbm.at[c, pl.ds(t * RPT, RPT)])


def _tc_transform_body(s_ref, e_ref, wg_ref, bg_ref, wb_ref, bb_ref,
                       ego_o_ref, norm_o_ref):
    s = jnp.concatenate([s_ref[0], s_ref[1]], axis=1)
    e = jnp.concatenate([e_ref[0], e_ref[1]], axis=1)
    z = jnp.dot(s, wg_ref[...], preferred_element_type=jnp.float32) + bg_ref[...]
    z = z + jnp.dot(e * s, wb_ref[...], preferred_element_type=jnp.float32) + bb_ref[...]
    y = jnp.where(z >= 0, z, 0.2 * z)
    nrm = jnp.sqrt(jnp.sum(y * y, axis=1, keepdims=True))
    norm_o_ref[...] = y / jnp.maximum(nrm, 1e-12)
    ego_o_ref[0] = y[:, :DH]
    ego_o_ref[1] = y[:, DH:]


_tc_transform = pl.pallas_call(
    _tc_transform_body,
    grid=(GRID,),
    in_specs=[
        pl.BlockSpec((NC, RB, DH), lambda i: (0, i, 0)),
        pl.BlockSpec((NC, RB, DH), lambda i: (0, i, 0)),
        pl.BlockSpec((D, D), lambda i: (0, 0)),
        pl.BlockSpec((1, D), lambda i: (0, 0)),
        pl.BlockSpec((D, D), lambda i: (0, 0)),
        pl.BlockSpec((1, D), lambda i: (0, 0)),
    ],
    out_specs=[
        pl.BlockSpec((NC, RB, DH), lambda i: (0, i, 0)),
        pl.BlockSpec((RB, D), lambda i: (i, 0)),
    ],
    out_shape=[
        jax.ShapeDtypeStruct((NC, NPAD, DH), jnp.float32),
        jax.ShapeDtypeStruct((NPAD, D), jnp.float32),
    ],
)


@functools.partial(
    pl.kernel,
    out_type=jax.ShapeDtypeStruct((4 * 2 * B, D), jnp.float32),
    mesh=_mesh,
    scratch_types=[
        pltpu.MemorySpace.VMEM((2, 128), jnp.int32),
        pltpu.MemorySpace.VMEM((4, 128, D), jnp.float32),
        pltpu.SemaphoreType.DMA,
    ],
    compiler_params=_sc_params,
)
def _final_gather(t0, t1, t2, t3, idx_hbm, out_hbm, idxb, gbuf, sem):
    c = lax.axis_index("c")
    t = lax.axis_index("s")
    wid = t * NC + c
    pltpu.sync_copy(idx_hbm.at[wid], idxb)
    nrows = 2 * B
    for j in range(2):
        gd = [pltpu.async_copy(tref.at[idxb.at[j]], gbuf.at[tab], sem)
              for tab, tref in enumerate((t0, t1, t2, t3))]
        for dd in gd:
            dd.wait()
        base = wid * 256 + j * 128
        for tab in range(4):
            pltpu.sync_copy(gbuf.at[tab],
                            out_hbm.at[pl.ds(tab * nrows + base, 128)])


def kernel(user_emb, item_emb, edge_values, W_gc, b_gc, W_bi, b_bi,
           edge_index, users, items):
    ego0 = jnp.concatenate([user_emb, item_emb], axis=0)
    ego_sp = ego0.reshape(NNODE, NC, DH).transpose(1, 0, 2)
    ego2 = jnp.pad(ego_sp, ((0, 0), (0, NPAD - NNODE), (0, 0)))

    row = edge_index[0].astype(jnp.int32)
    col = edge_index[1].astype(jnp.int32)
    colp = jnp.pad(col, (0, EP - E)).reshape(NS, NBLK, NSUB, 128)
    rowp = jnp.pad(row, (0, EP - E)).reshape(NS, NBLK, NSUB, 128)
    valp = jnp.pad(edge_values, (0, EP - E)).reshape(NS, NBLK, K)

    norm_tabs = []
    for k in range(L):
        side2 = _spmm(ego2.reshape(NC * NPAD, DH), colp, rowp, valp)
        ego2, norm_k = _tc_transform(side2, ego2, W_gc[k], b_gc[k],
                                     W_bi[k], b_bi[k])
        norm_tabs.append(norm_k)

    idx_all = jnp.concatenate(
        [users.astype(jnp.int32), items.astype(jnp.int32) + N_USER]
    ).reshape(32, 2, 128)
    out4 = _final_gather(ego0, norm_tabs[0], norm_tabs[1], norm_tabs[2],
                         idx_all)
    res = out4.reshape(4, 2 * B, D).transpose(1, 0, 2).reshape(2 * B, 4 * D)
    return res[:B], res[B:]


# speedup vs baseline: 6.7070x; 1.3317x over previous
import functools

import jax
import jax.numpy as jnp
from jax import lax
from jax.experimental import pallas as pl
from jax.experimental.pallas import tpu as pltpu
from jax.experimental.pallas import tpu_sc as plsc

N_USER = 25000
N_ITEM = 25000
NNODE = N_USER + N_ITEM
D = 64
DH = 32
L = 3
E = 800000
B = 4096

NC, NS = 2, 16
K = 256
NSUB = K // 128
NPAIR = 98
ET = NPAIR * 2 * K
EP = NS * ET
NPAD = 50176
RPT = NPAD // NS
RB = 512
GRID = NPAD // RB

_mesh = plsc.VectorSubcoreMesh(core_axis_name="c", subcore_axis_name="s")
_sc_params = pltpu.CompilerParams(use_tc_tiling_on_sc=False)


@functools.partial(
    pl.kernel,
    out_type=jax.ShapeDtypeStruct((NC, NPAD, DH), jnp.float32),
    mesh=_mesh,
    scratch_types=[
        pltpu.MemorySpace.VMEM_SHARED((NPAD, DH), jnp.float32),
        pltpu.MemorySpace.VMEM((2, 2 * NSUB, 128), jnp.int32),
        pltpu.MemorySpace.VMEM((2, 2 * NSUB, 128), jnp.int32),
        pltpu.MemorySpace.VMEM((2, 2 * K), jnp.float32),
        pltpu.MemorySpace.VMEM((2, K, DH), jnp.float32),
        pltpu.SemaphoreType.DMA((2,)),
        pltpu.SemaphoreType.DMA((2,)),
        pltpu.SemaphoreType.DMA,
    ],
    compiler_params=_sc_params,
)
def _spmm(tbl_hbm, rc_hbm, val_hbm, out_hbm,
          acc, colb, rowb, valb, gath, isem, gsem, ssem):
    c = lax.axis_index("c")
    t = lax.axis_index("s")

    def zrow(i, _):
        gath[0, i, pl.ds(0, 16)] = jnp.zeros((16,), jnp.float32)
        gath[0, i, pl.ds(16, 16)] = jnp.zeros((16,), jnp.float32)
        return 0
    lax.fori_loop(0, K, zrow, 0)
    zd = []
    for off in range(0, RPT, K):
        sz = min(K, RPT - off)
        zd.append(pltpu.async_copy(gath.at[0, pl.ds(0, sz)],
                                   acc.at[pl.ds(t * RPT + off, sz)], ssem))
    for dd in zd:
        dd.wait()

    pltpu.async_copy(rc_hbm.at[1, t, 0], colb.at[0], isem.at[0])
    pltpu.async_copy(rc_hbm.at[0, t, 0], rowb.at[0], isem.at[0])
    pltpu.async_copy(val_hbm.at[t, 0], valb.at[0], isem.at[0])
    plsc.subcore_barrier()

    off_c = c * NPAD

    def pair_body(i, _):
        for p in (0, 1):
            b = 2 * i + p
            pn = 1 - p
            pltpu.make_async_copy(rc_hbm.at[1, t, b], colb.at[p],
                                  isem.at[p]).wait()
            pltpu.make_async_copy(rc_hbm.at[0, t, b], rowb.at[p],
                                  isem.at[p]).wait()
            pltpu.make_async_copy(val_hbm.at[t, b], valb.at[p],
                                  isem.at[p]).wait()

            @pl.when(b + 1 < NPAIR)
            def _():
                pltpu.async_copy(rc_hbm.at[1, t, b + 1], colb.at[pn],
                                 isem.at[pn])
                pltpu.async_copy(rc_hbm.at[0, t, b + 1], rowb.at[pn],
                                 isem.at[pn])
                pltpu.async_copy(val_hbm.at[t, b + 1], valb.at[pn],
                                 isem.at[pn])

            def obody(ii, _):
                for j in range(2 * NSUB):
                    colb[p, j, pl.ds(ii * 16, 16)] = (
                        colb[p, j, pl.ds(ii * 16, 16)] + off_c)
                return 0
            lax.fori_loop(0, 8, obody, 0, unroll=True)

            g0 = [pltpu.async_copy(tbl_hbm.at[colb.at[p, j]],
                                   gath.at[0, pl.ds(j * 128, 128)],
                                   gsem.at[0])
                  for j in range(NSUB)]
            g1 = [pltpu.async_copy(tbl_hbm.at[colb.at[p, NSUB + j]],
                                   gath.at[1, pl.ds(j * 128, 128)],
                                   gsem.at[1])
                  for j in range(NSUB)]

            def scale(slot, vbase):
                def sbody(g, _):
                    vv = valb[p, pl.ds(vbase + g * 16, 16)]
                    base = g * 16
                    for j in range(16):
                        v = vv[j]
                        gath[slot, base + j, pl.ds(0, 16)] = (
                            gath[slot, base + j, pl.ds(0, 16)] * v)
                        gath[slot, base + j, pl.ds(16, 16)] = (
                            gath[slot, base + j, pl.ds(16, 16)] * v)
                    return 0
                lax.fori_loop(0, K // 16, sbody, 0)

            for dd in g0:
                dd.wait()
            scale(0, 0)
            s0 = [pltpu.async_copy(gath.at[0, pl.ds(j * 128, 128)],
                                   acc.at[rowb.at[p, j]], ssem, add=True)
                  for j in range(NSUB)]
            for dd in g1:
                dd.wait()
            scale(1, K)
            s1 = [pltpu.async_copy(gath.at[1, pl.ds(j * 128, 128)],
                                   acc.at[rowb.at[p, NSUB + j]], ssem,
                                   add=True)
                  for j in range(NSUB)]
            for dd in s0 + s1:
                dd.wait()
        return 0

    lax.fori_loop(0, NPAIR // 2, pair_body, 0)
    plsc.subcore_barrier()
    pltpu.sync_copy(acc.at[pl.ds(t * RPT, RPT)],
                    out_hbm.at[c, pl.ds(t * RPT, RPT)])


def _tc_transform_body(s_ref, e_ref, wg_ref, bg_ref, wb_ref, bb_ref,
                       ego_o_ref, norm_o_ref):
    s = jnp.concatenate([s_ref[0], s_ref[1]], axis=1)
    e = jnp.concatenate([e_ref[0], e_ref[1]], axis=1)
    z = jnp.dot(s, wg_ref[...], preferred_element_type=jnp.float32) + bg_ref[...]
    z = z + jnp.dot(e * s, wb_ref[...], preferred_element_type=jnp.float32) + bb_ref[...]
    y = jnp.where(z >= 0, z, 0.2 * z)
    nrm = jnp.sqrt(jnp.sum(y * y, axis=1, keepdims=True))
    norm_o_ref[...] = y / jnp.maximum(nrm, 1e-12)
    ego_o_ref[0] = y[:, :DH]
    ego_o_ref[1] = y[:, DH:]


_tc_transform = pl.pallas_call(
    _tc_transform_body,
    grid=(GRID,),
    in_specs=[
        pl.BlockSpec((NC, RB, DH), lambda i: (0, i, 0)),
        pl.BlockSpec((NC, RB, DH), lambda i: (0, i, 0)),
        pl.BlockSpec((D, D), lambda i: (0, 0)),
        pl.BlockSpec((1, D), lambda i: (0, 0)),
        pl.BlockSpec((D, D), lambda i: (0, 0)),
        pl.BlockSpec((1, D), lambda i: (0, 0)),
    ],
    out_specs=[
        pl.BlockSpec((NC, RB, DH), lambda i: (0, i, 0)),
        pl.BlockSpec((RB, D), lambda i: (i, 0)),
    ],
    out_shape=[
        jax.ShapeDtypeStruct((NC, NPAD, DH), jnp.float32),
        jax.ShapeDtypeStruct((NPAD, D), jnp.float32),
    ],
)


@functools.partial(
    pl.kernel,
    out_type=jax.ShapeDtypeStruct((4 * 2 * B, D), jnp.float32),
    mesh=_mesh,
    scratch_types=[
        pltpu.MemorySpace.VMEM((2, 128), jnp.int32),
        pltpu.MemorySpace.VMEM((4, 128, D), jnp.float32),
        pltpu.SemaphoreType.DMA,
    ],
    compiler_params=_sc_params,
)
def _final_gather(t0, t1, t2, t3, idx_hbm, out_hbm, idxb, gbuf, sem):
    c = lax.axis_index("c")
    t = lax.axis_index("s")
    wid = t * NC + c
    pltpu.sync_copy(idx_hbm.at[wid], idxb)
    nrows = 2 * B
    for j in range(2):
        gd = [pltpu.async_copy(tref.at[idxb.at[j]], gbuf.at[tab], sem)
              for tab, tref in enumerate((t0, t1, t2, t3))]
        for dd in gd:
            dd.wait()
        base = wid * 256 + j * 128
        for tab in range(4):
            pltpu.sync_copy(gbuf.at[tab],
                            out_hbm.at[pl.ds(tab * nrows + base, 128)])


def kernel(user_emb, item_emb, edge_values, W_gc, b_gc, W_bi, b_bi,
           edge_index, users, items):
    ego0 = jnp.concatenate([user_emb, item_emb], axis=0)
    ego_sp = ego0.reshape(NNODE, NC, DH).transpose(1, 0, 2)
    ego2 = jnp.pad(ego_sp, ((0, 0), (0, NPAD - NNODE), (0, 0)))

    rc = jnp.pad(edge_index.astype(jnp.int32),
                 ((0, 0), (0, EP - E))).reshape(2, NS, NPAIR, 2 * NSUB, 128)
    valp = jnp.pad(edge_values, (0, EP - E)).reshape(NS, NPAIR, 2 * K)

    norm_tabs = []
    for k in range(L):
        side2 = _spmm(ego2.reshape(NC * NPAD, DH), rc, valp)
        ego2, norm_k = _tc_transform(side2, ego2, W_gc[k], b_gc[k],
                                     W_bi[k], b_bi[k])
        norm_tabs.append(norm_k)

    idx_all = jnp.concatenate(
        [users.astype(jnp.int32), items.astype(jnp.int32) + N_USER]
    ).reshape(32, 2, 128)
    out4 = _final_gather(ego0, norm_tabs[0], norm_tabs[1], norm_tabs[2],
                         idx_all)
    res = out4.reshape(4, 2 * B, D).transpose(1, 0, 2).reshape(2 * B, 4 * D)
    return res[:B], res[B:]


# speedup vs baseline: 8.9746x; 1.3381x over previous
import functools

import jax
import jax.numpy as jnp
from jax import lax
from jax.experimental import pallas as pl
from jax.experimental.pallas import tpu as pltpu
from jax.experimental.pallas import tpu_sc as plsc

N_USER = 25000
N_ITEM = 25000
NNODE = N_USER + N_ITEM
D = 64
DH = 32
L = 3
E = 800000
B = 4096

NC, NS = 2, 16
K = 256
NSUB = K // 128
NPAIR = 98
ET = NPAIR * 2 * K
EP = NS * ET
NPAD = 50176
RPT = NPAD // NS
P4 = NPAD // 4
P2 = NPAD // 2
RP = 256
GRID = P4 // RP

_mesh = plsc.VectorSubcoreMesh(core_axis_name="c", subcore_axis_name="s")
_sc_params = pltpu.CompilerParams(use_tc_tiling_on_sc=False)


@functools.partial(
    pl.kernel,
    out_type=jax.ShapeDtypeStruct((NC, NPAD, DH), jnp.float32),
    mesh=_mesh,
    scratch_types=[
        pltpu.MemorySpace.VMEM_SHARED((NPAD, DH), jnp.float32),
        pltpu.MemorySpace.VMEM((2, 2 * NSUB, 128), jnp.int32),
        pltpu.MemorySpace.VMEM((2, 2 * NSUB, 128), jnp.int32),
        pltpu.MemorySpace.VMEM((2, 2 * K), jnp.float32),
        pltpu.MemorySpace.VMEM((2, K, DH), jnp.float32),
        pltpu.SemaphoreType.DMA((2,)),
        pltpu.SemaphoreType.DMA((2,)),
        pltpu.SemaphoreType.DMA,
    ],
    compiler_params=_sc_params,
)
def _spmm(tbl_hbm, rc_hbm, val_hbm, out_hbm,
          acc, colb, rowb, valb, gath, isem, gsem, ssem):
    c = lax.axis_index("c")
    t = lax.axis_index("s")

    def zrow(i, _):
        gath[0, i, pl.ds(0, 16)] = jnp.zeros((16,), jnp.float32)
        gath[0, i, pl.ds(16, 16)] = jnp.zeros((16,), jnp.float32)
        return 0
    lax.fori_loop(0, K, zrow, 0)
    zd = []
    for off in range(0, RPT, K):
        sz = min(K, RPT - off)
        zd.append(pltpu.async_copy(gath.at[0, pl.ds(0, sz)],
                                   acc.at[pl.ds(t * RPT + off, sz)], ssem))
    for dd in zd:
        dd.wait()

    pltpu.async_copy(rc_hbm.at[1, t, 0], colb.at[0], isem.at[0])
    pltpu.async_copy(rc_hbm.at[0, t, 0], rowb.at[0], isem.at[0])
    pltpu.async_copy(val_hbm.at[t, 0], valb.at[0], isem.at[0])
    plsc.subcore_barrier()

    off_c = c * NPAD

    def pair_body(i, _):
        for p in (0, 1):
            b = 2 * i + p
            pn = 1 - p
            pltpu.make_async_copy(rc_hbm.at[1, t, b], colb.at[p],
                                  isem.at[p]).wait()
            pltpu.make_async_copy(rc_hbm.at[0, t, b], rowb.at[p],
                                  isem.at[p]).wait()
            pltpu.make_async_copy(val_hbm.at[t, b], valb.at[p],
                                  isem.at[p]).wait()

            @pl.when(b + 1 < NPAIR)
            def _():
                pltpu.async_copy(rc_hbm.at[1, t, b + 1], colb.at[pn],
                                 isem.at[pn])
                pltpu.async_copy(rc_hbm.at[0, t, b + 1], rowb.at[pn],
                                 isem.at[pn])
                pltpu.async_copy(val_hbm.at[t, b + 1], valb.at[pn],
                                 isem.at[pn])

            def obody(ii, _):
                for j in range(2 * NSUB):
                    colb[p, j, pl.ds(ii * 16, 16)] = (
                        colb[p, j, pl.ds(ii * 16, 16)] + off_c)
                return 0
            lax.fori_loop(0, 8, obody, 0, unroll=True)

            g0 = [pltpu.async_copy(tbl_hbm.at[colb.at[p, j]],
                                   gath.at[0, pl.ds(j * 128, 128)],
                                   gsem.at[0])
                  for j in range(NSUB)]
            g1 = [pltpu.async_copy(tbl_hbm.at[colb.at[p, NSUB + j]],
                                   gath.at[1, pl.ds(j * 128, 128)],
                                   gsem.at[1])
                  for j in range(NSUB)]

            def scale(slot, vbase):
                def sbody(g, _):
                    vv = valb[p, pl.ds(vbase + g * 16, 16)]
                    base = g * 16
                    for j in range(16):
                        v = vv[j]
                        gath[slot, base + j, pl.ds(0, 16)] = (
                            gath[slot, base + j, pl.ds(0, 16)] * v)
                        gath[slot, base + j, pl.ds(16, 16)] = (
                            gath[slot, base + j, pl.ds(16, 16)] * v)
                    return 0
                lax.fori_loop(0, K // 16, sbody, 0)

            for dd in g0:
                dd.wait()
            scale(0, 0)
            s0 = [pltpu.async_copy(gath.at[0, pl.ds(j * 128, 128)],
                                   acc.at[rowb.at[p, j]], ssem, add=True)
                  for j in range(NSUB)]
            for dd in g1:
                dd.wait()
            scale(1, K)
            s1 = [pltpu.async_copy(gath.at[1, pl.ds(j * 128, 128)],
                                   acc.at[rowb.at[p, NSUB + j]], ssem,
                                   add=True)
                  for j in range(NSUB)]
            for dd in s0 + s1:
                dd.wait()
        return 0

    lax.fori_loop(0, NPAIR // 2, pair_body, 0)
    plsc.subcore_barrier()
    pltpu.sync_copy(acc.at[pl.ds(t * RPT, RPT)],
                    out_hbm.at[c, pl.ds(t * RPT, RPT)])


def _tc_transform_body(s_ref, e_ref, wg0_ref, wg1_ref, wb0_ref, wb1_ref,
                       bsum_ref, gg_ref, s0_ref, s1_ref,
                       ego_o_ref, norm_o_ref):
    s0, s1 = s_ref[0], s_ref[1]
    e0, e1 = e_ref[0], e_ref[1]
    f32 = jnp.float32
    z = (jnp.dot(s0, wg0_ref[...], preferred_element_type=f32)
         + jnp.dot(s1, wg1_ref[...], preferred_element_type=f32)
         + jnp.dot(e0 * s0, wb0_ref[...], preferred_element_type=f32)
         + jnp.dot(e1 * s1, wb1_ref[...], preferred_element_type=f32)
         + bsum_ref[...])
    y = jnp.where(z >= 0, z, 0.2 * z)
    nrmb = jnp.dot(y * y, gg_ref[...], preferred_element_type=f32)
    yn = y / jnp.maximum(jnp.sqrt(nrmb), 1e-12)
    norm_o_ref[...] = yn.reshape(2 * RP, 128)
    ego_o_ref[0] = jnp.dot(y, s0_ref[...], preferred_element_type=f32)
    ego_o_ref[1] = jnp.dot(y, s1_ref[...], preferred_element_type=f32)


_tc_transform = pl.pallas_call(
    _tc_transform_body,
    grid=(GRID,),
    in_specs=[
        pl.BlockSpec((NC, RP, 128), lambda i: (0, i, 0)),
        pl.BlockSpec((NC, RP, 128), lambda i: (0, i, 0)),
        pl.BlockSpec((128, 256), lambda i: (0, 0)),
        pl.BlockSpec((128, 256), lambda i: (0, 0)),
        pl.BlockSpec((128, 256), lambda i: (0, 0)),
        pl.BlockSpec((128, 256), lambda i: (0, 0)),
        pl.BlockSpec((1, 256), lambda i: (0, 0)),
        pl.BlockSpec((256, 256), lambda i: (0, 0)),
        pl.BlockSpec((256, 128), lambda i: (0, 0)),
        pl.BlockSpec((256, 128), lambda i: (0, 0)),
    ],
    out_specs=[
        pl.BlockSpec((NC, RP, 128), lambda i: (0, i, 0)),
        pl.BlockSpec((2 * RP, 128), lambda i: (i, 0)),
    ],
    out_shape=[
        jax.ShapeDtypeStruct((NC, P4, 128), jnp.float32),
        jax.ShapeDtypeStruct((P2, 128), jnp.float32),
    ],
)


@functools.partial(
    pl.kernel,
    out_type=jax.ShapeDtypeStruct((4 * 2 * B, D), jnp.float32),
    mesh=_mesh,
    scratch_types=[
        pltpu.MemorySpace.VMEM((2, 128), jnp.int32),
        pltpu.MemorySpace.VMEM((4, 128, D), jnp.float32),
        pltpu.SemaphoreType.DMA,
    ],
    compiler_params=_sc_params,
)
def _final_gather(t0, t1, t2, t3, idx_hbm, out_hbm, idxb, gbuf, sem):
    c = lax.axis_index("c")
    t = lax.axis_index("s")
    wid = t * NC + c
    pltpu.sync_copy(idx_hbm.at[wid], idxb)
    nrows = 2 * B
    for j in range(2):
        gd = [pltpu.async_copy(tref.at[idxb.at[j]], gbuf.at[tab], sem)
              for tab, tref in enumerate((t0, t1, t2, t3))]
        for dd in gd:
            dd.wait()
        base = wid * 256 + j * 128
        for tab in range(4):
            pltpu.sync_copy(gbuf.at[tab],
                            out_hbm.at[pl.ds(tab * nrows + base, 128)])


def kernel(user_emb, item_emb, edge_values, W_gc, b_gc, W_bi, b_bi,
           edge_index, users, items):
    ego0 = jnp.concatenate([user_emb, item_emb], axis=0)
    ego_sp = ego0.reshape(NNODE, NC, DH).transpose(1, 0, 2)
    ego_p = jnp.pad(ego_sp, ((0, 0), (0, NPAD - NNODE), (0, 0))
                    ).reshape(NC, P4, 128)
    ego0_pad = jnp.pad(ego0, ((0, NPAD - NNODE), (0, 0)))

    rc = jnp.pad(edge_index.astype(jnp.int32),
                 ((0, 0), (0, EP - E))).reshape(2, NS, NPAIR, 2 * NSUB, 128)
    valp = jnp.pad(edge_values, (0, EP - E)).reshape(NS, NPAIR, 2 * K)

    i4 = jnp.eye(4, dtype=jnp.float32)
    gg = jnp.kron(i4, jnp.ones((D, D), jnp.float32))
    sel0 = jnp.kron(i4, jnp.concatenate(
        [jnp.eye(DH, dtype=jnp.float32),
         jnp.zeros((DH, DH), jnp.float32)], axis=0))
    sel1 = jnp.kron(i4, jnp.concatenate(
        [jnp.zeros((DH, DH), jnp.float32),
         jnp.eye(DH, dtype=jnp.float32)], axis=0))

    norm_tabs = []
    for k in range(L):
        wg0 = jnp.kron(i4, W_gc[k][:DH])
        wg1 = jnp.kron(i4, W_gc[k][DH:])
        wb0 = jnp.kron(i4, W_bi[k][:DH])
        wb1 = jnp.kron(i4, W_bi[k][DH:])
        bsum = jnp.tile(b_gc[k] + b_bi[k], (1, 4))
        side_sp = _spmm(ego_p.reshape(NC * NPAD, DH), rc, valp)
        ego_p, norm_k = _tc_transform(side_sp.reshape(NC, P4, 128), ego_p,
                                      wg0, wg1, wb0, wb1,
                                      bsum, gg, sel0, sel1)
        norm_tabs.append(norm_k)

    idx_all = jnp.concatenate(
        [users.astype(jnp.int32), items.astype(jnp.int32) + N_USER]
    ).reshape(32, 2, 128)
    out4 = _final_gather(ego0_pad, norm_tabs[0].reshape(NPAD, D),
                         norm_tabs[1].reshape(NPAD, D),
                         norm_tabs[2].reshape(NPAD, D), idx_all)
    res = out4.reshape(4, 2 * B, D).transpose(1, 0, 2).reshape(2 * B, 4 * D)
    return res[:B], res[B:]


# speedup vs baseline: 9.2273x; 1.0282x over previous
import functools

import jax
import jax.numpy as jnp
from jax import lax
from jax.experimental import pallas as pl
from jax.experimental.pallas import tpu as pltpu
from jax.experimental.pallas import tpu_sc as plsc

N_USER = 25000
N_ITEM = 25000
NNODE = N_USER + N_ITEM
D = 64
DH = 32
L = 3
E = 800000
B = 4096

NC, NS = 2, 16
K = 256
NSUB = K // 128
NPAIR = 98
ET = NPAIR * 2 * K
EP = NS * ET
NPAD = 50176
RPT = NPAD // NS
P4 = NPAD // 4
P2 = NPAD // 2
RP = 256
GRID = P4 // RP

_mesh = plsc.VectorSubcoreMesh(core_axis_name="c", subcore_axis_name="s")
_sc_params = pltpu.CompilerParams(use_tc_tiling_on_sc=False)


@functools.partial(
    pl.kernel,
    out_type=jax.ShapeDtypeStruct((NC, NPAD, DH), jnp.float32),
    mesh=_mesh,
    scratch_types=[
        pltpu.MemorySpace.VMEM_SHARED((NPAD, DH), jnp.float32),
        pltpu.MemorySpace.VMEM((2, 2 * NSUB, 128), jnp.int32),
        pltpu.MemorySpace.VMEM((2, 2 * NSUB, 128), jnp.int32),
        pltpu.MemorySpace.VMEM((2, 2 * K), jnp.float32),
        pltpu.MemorySpace.VMEM((2, K, DH), jnp.float32),
        pltpu.SemaphoreType.DMA((2,)),
        pltpu.SemaphoreType.DMA((2,)),
        pltpu.SemaphoreType.DMA,
    ],
    compiler_params=_sc_params,
)
def _spmm(tbl_hbm, rc_hbm, val_hbm, out_hbm,
          acc, colb, rowb, valb, gath, isem, gsem, ssem):
    c = lax.axis_index("c")
    t = lax.axis_index("s")

    def zrow(i, _):
        gath[0, i, pl.ds(0, 16)] = jnp.zeros((16,), jnp.float32)
        gath[0, i, pl.ds(16, 16)] = jnp.zeros((16,), jnp.float32)
        return 0
    lax.fori_loop(0, K, zrow, 0)
    zd = []
    for off in range(0, RPT, K):
        sz = min(K, RPT - off)
        zd.append(pltpu.async_copy(gath.at[0, pl.ds(0, sz)],
                                   acc.at[pl.ds(t * RPT + off, sz)], ssem))
    for dd in zd:
        dd.wait()

    pltpu.async_copy(rc_hbm.at[1, t, 0], colb.at[0], isem.at[0])
    pltpu.async_copy(rc_hbm.at[0, t, 0], rowb.at[0], isem.at[0])
    pltpu.async_copy(val_hbm.at[t, 0], valb.at[0], isem.at[0])
    plsc.subcore_barrier()

    off_c = c

    def pair_body(i, _):
        for p in (0, 1):
            b = 2 * i + p
            pn = 1 - p
            pltpu.make_async_copy(rc_hbm.at[1, t, b], colb.at[p],
                                  isem.at[p]).wait()
            pltpu.make_async_copy(rc_hbm.at[0, t, b], rowb.at[p],
                                  isem.at[p]).wait()
            pltpu.make_async_copy(val_hbm.at[t, b], valb.at[p],
                                  isem.at[p]).wait()

            @pl.when(b + 1 < NPAIR)
            def _():
                pltpu.async_copy(rc_hbm.at[1, t, b + 1], colb.at[pn],
                                 isem.at[pn])
                pltpu.async_copy(rc_hbm.at[0, t, b + 1], rowb.at[pn],
                                 isem.at[pn])
                pltpu.async_copy(val_hbm.at[t, b + 1], valb.at[pn],
                                 isem.at[pn])

            def obody(ii, _):
                for j in range(2 * NSUB):
                    colb[p, j, pl.ds(ii * 16, 16)] = (
                        colb[p, j, pl.ds(ii * 16, 16)] * 2 + off_c)
                return 0
            lax.fori_loop(0, 8, obody, 0, unroll=True)

            g0 = [pltpu.async_copy(tbl_hbm.at[colb.at[p, j]],
                                   gath.at[0, pl.ds(j * 128, 128)],
                                   gsem.at[0])
                  for j in range(NSUB)]
            g1 = [pltpu.async_copy(tbl_hbm.at[colb.at[p, NSUB + j]],
                                   gath.at[1, pl.ds(j * 128, 128)],
                                   gsem.at[1])
                  for j in range(NSUB)]

            def scale(slot, vbase):
                def sbody(g, _):
                    vv = valb[p, pl.ds(vbase + g * 16, 16)]
                    base = g * 16
                    for j in range(16):
                        v = vv[j]
                        gath[slot, base + j, pl.ds(0, 16)] = (
                            gath[slot, base + j, pl.ds(0, 16)] * v)
                        gath[slot, base + j, pl.ds(16, 16)] = (
                            gath[slot, base + j, pl.ds(16, 16)] * v)
                    return 0
                lax.fori_loop(0, K // 16, sbody, 0)

            for dd in g0:
                dd.wait()
            scale(0, 0)
            s0 = [pltpu.async_copy(gath.at[0, pl.ds(j * 128, 128)],
                                   acc.at[rowb.at[p, j]], ssem, add=True)
                  for j in range(NSUB)]
            for dd in g1:
                dd.wait()
            scale(1, K)
            s1 = [pltpu.async_copy(gath.at[1, pl.ds(j * 128, 128)],
                                   acc.at[rowb.at[p, NSUB + j]], ssem,
                                   add=True)
                  for j in range(NSUB)]
            for dd in s0 + s1:
                dd.wait()
        return 0

    lax.fori_loop(0, NPAIR // 2, pair_body, 0)
    plsc.subcore_barrier()
    pltpu.sync_copy(acc.at[pl.ds(t * RPT, RPT)],
                    out_hbm.at[c, pl.ds(t * RPT, RPT)])


def _tc_transform_body(s_ref, e_ref, pa_ref, pb_ref, wg_ref, wb_ref,
                       b2_ref, gg_ref, ego_o_ref, norm_o_ref):
    f32 = jnp.float32
    snm = (jnp.dot(s_ref[0], pa_ref[...], preferred_element_type=f32)
           + jnp.dot(s_ref[1], pb_ref[...], preferred_element_type=f32)
           ).reshape(2 * RP, 128)
    e = e_ref[...]
    z = (jnp.dot(snm, wg_ref[...], preferred_element_type=f32)
         + jnp.dot(e * snm, wb_ref[...], preferred_element_type=f32)
         + b2_ref[...])
    y = jnp.where(z >= 0, z, 0.2 * z)
    nrmb = jnp.dot(y * y, gg_ref[...], preferred_element_type=f32)
    norm_o_ref[...] = y / jnp.maximum(jnp.sqrt(nrmb), 1e-12)
    ego_o_ref[...] = y


_tc_transform = pl.pallas_call(
    _tc_transform_body,
    grid=(GRID,),
    in_specs=[
        pl.BlockSpec((NC, RP, 128), lambda i: (0, i, 0)),
        pl.BlockSpec((2 * RP, 128), lambda i: (i, 0)),
        pl.BlockSpec((128, 256), lambda i: (0, 0)),
        pl.BlockSpec((128, 256), lambda i: (0, 0)),
        pl.BlockSpec((128, 128), lambda i: (0, 0)),
        pl.BlockSpec((128, 128), lambda i: (0, 0)),
        pl.BlockSpec((1, 128), lambda i: (0, 0)),
        pl.BlockSpec((128, 128), lambda i: (0, 0)),
    ],
    out_specs=[
        pl.BlockSpec((2 * RP, 128), lambda i: (i, 0)),
        pl.BlockSpec((2 * RP, 128), lambda i: (i, 0)),
    ],
    out_shape=[
        jax.ShapeDtypeStruct((P2, 128), jnp.float32),
        jax.ShapeDtypeStruct((P2, 128), jnp.float32),
    ],
)


@functools.partial(
    pl.kernel,
    out_type=jax.ShapeDtypeStruct((4 * 2 * B, D), jnp.float32),
    mesh=_mesh,
    scratch_types=[
        pltpu.MemorySpace.VMEM((2, 128), jnp.int32),
        pltpu.MemorySpace.VMEM((4, 128, D), jnp.float32),
        pltpu.SemaphoreType.DMA,
    ],
    compiler_params=_sc_params,
)
def _final_gather(t0, t1, t2, t3, idx_hbm, out_hbm, idxb, gbuf, sem):
    c = lax.axis_index("c")
    t = lax.axis_index("s")
    wid = t * NC + c
    pltpu.sync_copy(idx_hbm.at[wid], idxb)
    nrows = 2 * B
    for j in range(2):
        gd = [pltpu.async_copy(tref.at[idxb.at[j]], gbuf.at[tab], sem)
              for tab, tref in enumerate((t0, t1, t2, t3))]
        for dd in gd:
            dd.wait()
        base = wid * 256 + j * 128
        for tab in range(4):
            pltpu.sync_copy(gbuf.at[tab],
                            out_hbm.at[pl.ds(tab * nrows + base, 128)])


def kernel(user_emb, item_emb, edge_values, W_gc, b_gc, W_bi, b_bi,
           edge_index, users, items):
    ego0 = jnp.concatenate([user_emb, item_emb], axis=0)
    ego0_pad = jnp.pad(ego0, ((0, NPAD - NNODE), (0, 0)))
    ego_nm = ego0_pad.reshape(P2, 128)

    rc = jnp.pad(edge_index.astype(jnp.int32),
                 ((0, 0), (0, EP - E))).reshape(2, NS, NPAIR, 2 * NSUB, 128)
    valp = jnp.pad(edge_values, (0, EP - E)).reshape(NS, NPAIR, 2 * K)

    f32 = jnp.float32
    i2 = jnp.eye(2, dtype=f32)
    i4 = jnp.eye(4, dtype=f32)
    gg = jnp.kron(i2, jnp.ones((D, D), f32))
    h0 = jnp.concatenate([jnp.eye(DH, dtype=f32),
                          jnp.zeros((DH, DH), f32)], axis=1)
    h1 = jnp.concatenate([jnp.zeros((DH, DH), f32),
                          jnp.eye(DH, dtype=f32)], axis=1)
    pa = jnp.kron(i4, h0)
    pb = jnp.kron(i4, h1)

    norm_tabs = []
    for k in range(L):
        wg = jnp.kron(i2, W_gc[k])
        wb = jnp.kron(i2, W_bi[k])
        b2 = jnp.tile(b_gc[k] + b_bi[k], (1, 2))
        side_sp = _spmm(ego_nm.reshape(2 * NPAD, DH), rc, valp)
        ego_nm, norm_k = _tc_transform(side_sp.reshape(NC, P4, 128), ego_nm,
                                       pa, pb, wg, wb, b2, gg)
        norm_tabs.append(norm_k)

    idx_all = jnp.concatenate(
        [users.astype(jnp.int32), items.astype(jnp.int32) + N_USER]
    ).reshape(32, 2, 128)
    out4 = _final_gather(ego0_pad, norm_tabs[0].reshape(NPAD, D),
                         norm_tabs[1].reshape(NPAD, D),
                         norm_tabs[2].reshape(NPAD, D), idx_all)
    res = out4.reshape(4, 2 * B, D).transpose(1, 0, 2).reshape(2 * B, 4 * D)
    return res[:B], res[B:]


# speedup vs baseline: 9.8814x; 1.0709x over previous
import functools

import jax
import jax.numpy as jnp
from jax import lax
from jax.experimental import pallas as pl
from jax.experimental.pallas import tpu as pltpu
from jax.experimental.pallas import tpu_sc as plsc

N_USER = 25000
N_ITEM = 25000
NNODE = N_USER + N_ITEM
D = 64
DH = 32
L = 3
E = 800000
B = 4096

NC, NS = 2, 16
K = 256
NSUB = K // 128
NPAIR = 98
ET = NPAIR * 2 * K
EP = NS * ET
NPAD = 50176
RPT = NPAD // NS
P4 = NPAD // 4
P2 = NPAD // 2
RP = 1792
GRID = P4 // RP

_mesh = plsc.VectorSubcoreMesh(core_axis_name="c", subcore_axis_name="s")
_sc_params = pltpu.CompilerParams(use_tc_tiling_on_sc=False)


@functools.partial(
    pl.kernel,
    out_type=jax.ShapeDtypeStruct((NC, NPAD, DH), jnp.float32),
    mesh=_mesh,
    scratch_types=[
        pltpu.MemorySpace.VMEM_SHARED((NPAD, DH), jnp.float32),
        pltpu.MemorySpace.VMEM((2, 2 * NSUB, 128), jnp.int32),
        pltpu.MemorySpace.VMEM((2, 2 * NSUB, 128), jnp.int32),
        pltpu.MemorySpace.VMEM((2, 2 * K), jnp.float32),
        pltpu.MemorySpace.VMEM((2, K, DH), jnp.float32),
        pltpu.SemaphoreType.DMA((2,)),
        pltpu.SemaphoreType.DMA((2,)),
        pltpu.SemaphoreType.DMA,
    ],
    compiler_params=_sc_params,
)
def _spmm(tbl_hbm, rc_hbm, val_hbm, out_hbm,
          acc, colb, rowb, valb, gath, isem, gsem, ssem):
    c = lax.axis_index("c")
    t = lax.axis_index("s")

    def zrow(i, _):
        gath[0, i, pl.ds(0, 16)] = jnp.zeros((16,), jnp.float32)
        gath[0, i, pl.ds(16, 16)] = jnp.zeros((16,), jnp.float32)
        return 0
    lax.fori_loop(0, K, zrow, 0)
    zd = []
    for off in range(0, RPT, K):
        sz = min(K, RPT - off)
        zd.append(pltpu.async_copy(gath.at[0, pl.ds(0, sz)],
                                   acc.at[pl.ds(t * RPT + off, sz)], ssem))
    for dd in zd:
        dd.wait()

    pltpu.async_copy(rc_hbm.at[1, t, 0], colb.at[0], isem.at[0])
    pltpu.async_copy(rc_hbm.at[0, t, 0], rowb.at[0], isem.at[0])
    pltpu.async_copy(val_hbm.at[t, 0], valb.at[0], isem.at[0])
    plsc.subcore_barrier()

    off_c = c

    def pair_body(i, _):
        for p in (0, 1):
            b = 2 * i + p
            pn = 1 - p
            pltpu.make_async_copy(rc_hbm.at[1, t, b], colb.at[p],
                                  isem.at[p]).wait()
            pltpu.make_async_copy(rc_hbm.at[0, t, b], rowb.at[p],
                                  isem.at[p]).wait()
            pltpu.make_async_copy(val_hbm.at[t, b], valb.at[p],
                                  isem.at[p]).wait()

            @pl.when(b + 1 < NPAIR)
            def _():
                pltpu.async_copy(rc_hbm.at[1, t, b + 1], colb.at[pn],
                                 isem.at[pn])
                pltpu.async_copy(rc_hbm.at[0, t, b + 1], rowb.at[pn],
                                 isem.at[pn])
                pltpu.async_copy(val_hbm.at[t, b + 1], valb.at[pn],
                                 isem.at[pn])

            def obody(ii, _):
                for j in range(2 * NSUB):
                    colb[p, j, pl.ds(ii * 16, 16)] = (
                        colb[p, j, pl.ds(ii * 16, 16)] * 2 + off_c)
                return 0
            lax.fori_loop(0, 8, obody, 0, unroll=True)

            g0 = [pltpu.async_copy(tbl_hbm.at[colb.at[p, j]],
                                   gath.at[0, pl.ds(j * 128, 128)],
                                   gsem.at[0])
                  for j in range(NSUB)]
            g1 = [pltpu.async_copy(tbl_hbm.at[colb.at[p, NSUB + j]],
                                   gath.at[1, pl.ds(j * 128, 128)],
                                   gsem.at[1])
                  for j in range(NSUB)]

            def scale(slot, vbase):
                def sbody(g, _):
                    vv = valb[p, pl.ds(vbase + g * 16, 16)]
                    base = g * 16
                    for j in range(16):
                        v = vv[j]
                        gath[slot, base + j, pl.ds(0, 16)] = (
                            gath[slot, base + j, pl.ds(0, 16)] * v)
                        gath[slot, base + j, pl.ds(16, 16)] = (
                            gath[slot, base + j, pl.ds(16, 16)] * v)
                    return 0
                lax.fori_loop(0, K // 16, sbody, 0)

            for dd in g0:
                dd.wait()
            scale(0, 0)
            s0 = [pltpu.async_copy(gath.at[0, pl.ds(j * 128, 128)],
                                   acc.at[rowb.at[p, j]], ssem, add=True)
                  for j in range(NSUB)]
            for dd in g1:
                dd.wait()
            scale(1, K)
            s1 = [pltpu.async_copy(gath.at[1, pl.ds(j * 128, 128)],
                                   acc.at[rowb.at[p, NSUB + j]], ssem,
                                   add=True)
                  for j in range(NSUB)]
            for dd in s0 + s1:
                dd.wait()
        return 0

    lax.fori_loop(0, NPAIR // 2, pair_body, 0)
    plsc.subcore_barrier()
    pltpu.sync_copy(acc.at[pl.ds(t * RPT, RPT)],
                    out_hbm.at[c, pl.ds(t * RPT, RPT)])


def _tc_transform_body(s_ref, e_ref, pa_ref, pb_ref, wg_ref, wb_ref,
                       b2_ref, gg_ref, ego_o_ref, norm_o_ref):
    f32 = jnp.float32
    snm = (jnp.dot(s_ref[0], pa_ref[...], preferred_element_type=f32)
           + jnp.dot(s_ref[1], pb_ref[...], preferred_element_type=f32)
           ).reshape(2 * RP, 128)
    e = e_ref[...]
    z = (jnp.dot(snm, wg_ref[...], preferred_element_type=f32)
         + jnp.dot(e * snm, wb_ref[...], preferred_element_type=f32)
         + b2_ref[...])
    y = jnp.where(z >= 0, z, 0.2 * z)
    nrmb = jnp.dot(y * y, gg_ref[...], preferred_element_type=f32)
    norm_o_ref[...] = y / jnp.maximum(jnp.sqrt(nrmb), 1e-12)
    ego_o_ref[...] = y


_tc_transform = pl.pallas_call(
    _tc_transform_body,
    grid=(GRID,),
    in_specs=[
        pl.BlockSpec((NC, RP, 128), lambda i: (0, i, 0)),
        pl.BlockSpec((2 * RP, 128), lambda i: (i, 0)),
        pl.BlockSpec((128, 256), lambda i: (0, 0)),
        pl.BlockSpec((128, 256), lambda i: (0, 0)),
        pl.BlockSpec((128, 128), lambda i: (0, 0)),
        pl.BlockSpec((128, 128), lambda i: (0, 0)),
        pl.BlockSpec((1, 128), lambda i: (0, 0)),
        pl.BlockSpec((128, 128), lambda i: (0, 0)),
    ],
    out_specs=[
        pl.BlockSpec((2 * RP, 128), lambda i: (i, 0)),
        pl.BlockSpec((2 * RP, 128), lambda i: (i, 0)),
    ],
    out_shape=[
        jax.ShapeDtypeStruct((P2, 128), jnp.float32),
        jax.ShapeDtypeStruct((P2, 128), jnp.float32),
    ],
)


@functools.partial(
    pl.kernel,
    out_type=jax.ShapeDtypeStruct((4 * 2 * B, D), jnp.float32),
    mesh=_mesh,
    scratch_types=[
        pltpu.MemorySpace.VMEM((2, 128), jnp.int32),
        pltpu.MemorySpace.VMEM((4, 128, D), jnp.float32),
        pltpu.SemaphoreType.DMA,
    ],
    compiler_params=_sc_params,
)
def _final_gather(t0, t1, t2, t3, idx_hbm, out_hbm, idxb, gbuf, sem):
    c = lax.axis_index("c")
    t = lax.axis_index("s")
    wid = t * NC + c
    pltpu.sync_copy(idx_hbm.at[wid], idxb)
    nrows = 2 * B
    for j in range(2):
        gd = [pltpu.async_copy(tref.at[idxb.at[j]], gbuf.at[tab], sem)
              for tab, tref in enumerate((t0, t1, t2, t3))]
        for dd in gd:
            dd.wait()
        base = wid * 256 + j * 128
        for tab in range(4):
            pltpu.sync_copy(gbuf.at[tab],
                            out_hbm.at[pl.ds(tab * nrows + base, 128)])


def kernel(user_emb, item_emb, edge_values, W_gc, b_gc, W_bi, b_bi,
           edge_index, users, items):
    ego0 = jnp.concatenate([user_emb, item_emb], axis=0)
    ego0_pad = jnp.pad(ego0, ((0, NPAD - NNODE), (0, 0)))
    ego_nm = ego0_pad.reshape(P2, 128)

    rc = jnp.pad(edge_index.astype(jnp.int32),
                 ((0, 0), (0, EP - E))).reshape(2, NS, NPAIR, 2 * NSUB, 128)
    valp = jnp.pad(edge_values, (0, EP - E)).reshape(NS, NPAIR, 2 * K)

    f32 = jnp.float32
    i2 = jnp.eye(2, dtype=f32)
    i4 = jnp.eye(4, dtype=f32)
    gg = jnp.kron(i2, jnp.ones((D, D), f32))
    h0 = jnp.concatenate([jnp.eye(DH, dtype=f32),
                          jnp.zeros((DH, DH), f32)], axis=1)
    h1 = jnp.concatenate([jnp.zeros((DH, DH), f32),
                          jnp.eye(DH, dtype=f32)], axis=1)
    pa = jnp.kron(i4, h0)
    pb = jnp.kron(i4, h1)

    norm_tabs = []
    for k in range(L):
        wg = jnp.kron(i2, W_gc[k])
        wb = jnp.kron(i2, W_bi[k])
        b2 = jnp.tile(b_gc[k] + b_bi[k], (1, 2))
        side_sp = _spmm(ego_nm.reshape(2 * NPAD, DH), rc, valp)
        ego_nm, norm_k = _tc_transform(side_sp.reshape(NC, P4, 128), ego_nm,
                                       pa, pb, wg, wb, b2, gg)
        norm_tabs.append(norm_k)

    idx_all = jnp.concatenate(
        [users.astype(jnp.int32), items.astype(jnp.int32) + N_USER]
    ).reshape(32, 2, 128)
    out4 = _final_gather(ego0_pad, norm_tabs[0].reshape(NPAD, D),
                         norm_tabs[1].reshape(NPAD, D),
                         norm_tabs[2].reshape(NPAD, D), idx_all)
    res = out4.reshape(4, 2 * B, D).transpose(1, 0, 2).reshape(2 * B, 4 * D)
    return res[:B], res[B:]


# speedup vs baseline: 11.8461x; 1.1988x over previous
import functools

import jax
import jax.numpy as jnp
from jax import lax
from jax.experimental import pallas as pl
from jax.experimental.pallas import tpu as pltpu
from jax.experimental.pallas import tpu_sc as plsc

N_USER = 25000
N_ITEM = 25000
NNODE = N_USER + N_ITEM
D = 64
DH = 32
L = 3
E = 800000
B = 4096

NC, NS = 2, 16
K = 256
NSUB = K // 128
NBLK = 196
ET = NBLK * K
EP = NS * ET
NPAD = 50176
RPT = NPAD // NS
P4 = NPAD // 4
P2 = NPAD // 2
RP = 1792
GRID = P4 // RP

_mesh = plsc.VectorSubcoreMesh(core_axis_name="c", subcore_axis_name="s")
_sc_params = pltpu.CompilerParams(use_tc_tiling_on_sc=False)


@functools.partial(
    pl.kernel,
    out_type=jax.ShapeDtypeStruct((NC, NPAD, DH), jnp.float32),
    mesh=_mesh,
    scratch_types=[
        pltpu.MemorySpace.VMEM_SHARED((NPAD, DH), jnp.float32),
        pltpu.MemorySpace.VMEM((4, NSUB, 128), jnp.int32),
        pltpu.MemorySpace.VMEM((4, NSUB, 128), jnp.int32),
        pltpu.MemorySpace.VMEM((4, K), jnp.float32),
        pltpu.MemorySpace.VMEM((2, NSUB, 128), jnp.int32),
        pltpu.MemorySpace.VMEM((2, K, DH), jnp.float32),
        pltpu.SemaphoreType.DMA((4,)),
        pltpu.SemaphoreType.DMA((2,)),
        pltpu.SemaphoreType.DMA((2,)),
    ],
    compiler_params=_sc_params,
)
def _spmm(tbl_hbm, rc_hbm, val_hbm, out_hbm,
          acc, colb, rowb, valb, rowS, gath, isem, gsem, ssem):
    c = lax.axis_index("c")
    t = lax.axis_index("s")

    def zrow(i, _):
        gath[0, i, pl.ds(0, 16)] = jnp.zeros((16,), jnp.float32)
        gath[0, i, pl.ds(16, 16)] = jnp.zeros((16,), jnp.float32)
        return 0
    lax.fori_loop(0, K, zrow, 0)
    zd = []
    for off in range(0, RPT, K):
        sz = min(K, RPT - off)
        zd.append(pltpu.async_copy(gath.at[0, pl.ds(0, sz)],
                                   acc.at[pl.ds(t * RPT + off, sz)],
                                   ssem.at[0]))
    for dd in zd:
        dd.wait()

    off_c = c

    def issue_idx(b, r):
        pltpu.async_copy(rc_hbm.at[1, t, b], colb.at[r], isem.at[r])
        pltpu.async_copy(rc_hbm.at[0, t, b], rowb.at[r], isem.at[r])
        pltpu.async_copy(val_hbm.at[t, b], valb.at[r], isem.at[r])

    def drain_idx(b, r):
        pltpu.make_async_copy(rc_hbm.at[1, t, b], colb.at[r],
                              isem.at[r]).wait()
        pltpu.make_async_copy(rc_hbm.at[0, t, b], rowb.at[r],
                              isem.at[r]).wait()
        pltpu.make_async_copy(val_hbm.at[t, b], valb.at[r],
                              isem.at[r]).wait()

    def drain_scat(s):
        for j in range(NSUB):
            pltpu.make_async_copy(gath.at[s, pl.ds(j * 128, 128)],
                                  acc.at[rowS.at[s, j]], ssem.at[s]).wait()

    def produce(b, r, s, first):
        drain_idx(b, r)

        def obody(ii, _):
            for j in range(NSUB):
                colb[r, j, pl.ds(ii * 16, 16)] = (
                    colb[r, j, pl.ds(ii * 16, 16)] * 2 + off_c)
            return 0
        lax.fori_loop(0, 8, obody, 0, unroll=True)

        def after_credit():
            for j in range(NSUB):
                def cbody(ii, _):
                    rowS[s, j, pl.ds(ii * 16, 16)] = rowb[r, j,
                                                          pl.ds(ii * 16, 16)]
                    return 0
                lax.fori_loop(0, 8, cbody, 0, unroll=True)

        if first:
            @pl.when(b >= 2)
            def _():
                drain_scat(s)
        else:
            drain_scat(s)
        after_credit()
        return [pltpu.async_copy(tbl_hbm.at[colb.at[r, j]],
                                 gath.at[s, pl.ds(j * 128, 128)], gsem.at[s])
                for j in range(NSUB)]

    def scale(s, r):
        def sbody(g, _):
            vv = valb[r, pl.ds(g * 16, 16)]
            base = g * 16
            for j in range(16):
                v = vv[j]
                gath[s, base + j, pl.ds(0, 16)] = (
                    gath[s, base + j, pl.ds(0, 16)] * v)
                gath[s, base + j, pl.ds(16, 16)] = (
                    gath[s, base + j, pl.ds(16, 16)] * v)
            return 0
        lax.fori_loop(0, K // 16, sbody, 0)

    def consume(x, r, s):
        pltpu.make_async_copy(tbl_hbm.at[colb.at[r, 0]],
                              gath.at[s, pl.ds(0, 128)], gsem.at[s]).wait()
        pltpu.make_async_copy(tbl_hbm.at[colb.at[r, 1]],
                              gath.at[s, pl.ds(128, 128)], gsem.at[s]).wait()
        scale(s, r)
        for j in range(NSUB):
            pltpu.async_copy(gath.at[s, pl.ds(j * 128, 128)],
                             acc.at[rowS.at[s, j]], ssem.at[s], add=True)

    for bb in range(3):
        issue_idx(bb, bb)
    plsc.subcore_barrier()

    NB = ET // K

    def quad_body(o, _):
        for u in range(4):
            b = 4 * o + u
            s = u % 2
            gd = produce(b, u, s, first=(u < 2))
            xu = (u - 1) % 4
            xs = (u - 1) % 2
            pref = b + 2
            if u == 0:
                @pl.when(o > 0)
                def _():
                    consume(b - 1, xu, xs)
                    issue_idx(pref, (u + 2) % 4)
            elif u == 1:
                consume(b - 1, xu, xs)
                issue_idx(pref, (u + 2) % 4)
            else:
                consume(b - 1, xu, xs)

                @pl.when(o < NB // 4 - 1)
                def _():
                    issue_idx(pref, (u + 2) % 4)
        return 0

    lax.fori_loop(0, NB // 4, quad_body, 0)
    consume(NB - 1, (NB - 1) % 4, (NB - 1) % 2)
    drain_scat(0)
    drain_scat(1)
    plsc.subcore_barrier()
    pltpu.sync_copy(acc.at[pl.ds(t * RPT, RPT)],
                    out_hbm.at[c, pl.ds(t * RPT, RPT)])


def _tc_transform_body(s_ref, e_ref, pa_ref, pb_ref, wg_ref, wb_ref,
                       b2_ref, gg_ref, ego_o_ref, norm_o_ref):
    f32 = jnp.float32
    snm = (jnp.dot(s_ref[0], pa_ref[...], preferred_element_type=f32)
           + jnp.dot(s_ref[1], pb_ref[...], preferred_element_type=f32)
           ).reshape(2 * RP, 128)
    e = e_ref[...]
    z = (jnp.dot(snm, wg_ref[...], preferred_element_type=f32)
         + jnp.dot(e * snm, wb_ref[...], preferred_element_type=f32)
         + b2_ref[...])
    y = jnp.where(z >= 0, z, 0.2 * z)
    nrmb = jnp.dot(y * y, gg_ref[...], preferred_element_type=f32)
    norm_o_ref[...] = y / jnp.maximum(jnp.sqrt(nrmb), 1e-12)
    ego_o_ref[...] = y


_tc_transform = pl.pallas_call(
    _tc_transform_body,
    grid=(GRID,),
    in_specs=[
        pl.BlockSpec((NC, RP, 128), lambda i: (0, i, 0)),
        pl.BlockSpec((2 * RP, 128), lambda i: (i, 0)),
        pl.BlockSpec((128, 256), lambda i: (0, 0)),
        pl.BlockSpec((128, 256), lambda i: (0, 0)),
        pl.BlockSpec((128, 128), lambda i: (0, 0)),
        pl.BlockSpec((128, 128), lambda i: (0, 0)),
        pl.BlockSpec((1, 128), lambda i: (0, 0)),
        pl.BlockSpec((128, 128), lambda i: (0, 0)),
    ],
    out_specs=[
        pl.BlockSpec((2 * RP, 128), lambda i: (i, 0)),
        pl.BlockSpec((2 * RP, 128), lambda i: (i, 0)),
    ],
    out_shape=[
        jax.ShapeDtypeStruct((P2, 128), jnp.float32),
        jax.ShapeDtypeStruct((P2, 128), jnp.float32),
    ],
)


@functools.partial(
    pl.kernel,
    out_type=jax.ShapeDtypeStruct((4 * 2 * B, D), jnp.float32),
    mesh=_mesh,
    scratch_types=[
        pltpu.MemorySpace.VMEM((2, 128), jnp.int32),
        pltpu.MemorySpace.VMEM((4, 128, D), jnp.float32),
        pltpu.SemaphoreType.DMA,
    ],
    compiler_params=_sc_params,
)
def _final_gather(t0, t1, t2, t3, idx_hbm, out_hbm, idxb, gbuf, sem):
    c = lax.axis_index("c")
    t = lax.axis_index("s")
    wid = t * NC + c
    pltpu.sync_copy(idx_hbm.at[wid], idxb)
    nrows = 2 * B
    for j in range(2):
        gd = [pltpu.async_copy(tref.at[idxb.at[j]], gbuf.at[tab], sem)
              for tab, tref in enumerate((t0, t1, t2, t3))]
        for dd in gd:
            dd.wait()
        base = wid * 256 + j * 128
        for tab in range(4):
            pltpu.sync_copy(gbuf.at[tab],
                            out_hbm.at[pl.ds(tab * nrows + base, 128)])


def kernel(user_emb, item_emb, edge_values, W_gc, b_gc, W_bi, b_bi,
           edge_index, users, items):
    ego0 = jnp.concatenate([user_emb, item_emb], axis=0)
    ego0_pad = jnp.pad(ego0, ((0, NPAD - NNODE), (0, 0)))
    ego_nm = ego0_pad.reshape(P2, 128)

    rc = jnp.pad(edge_index.astype(jnp.int32),
                 ((0, 0), (0, EP - E))).reshape(2, NS, NBLK, NSUB, 128)
    valp = jnp.pad(edge_values, (0, EP - E)).reshape(NS, NBLK, K)

    f32 = jnp.float32
    i2 = jnp.eye(2, dtype=f32)
    i4 = jnp.eye(4, dtype=f32)
    gg = jnp.kron(i2, jnp.ones((D, D), f32))
    h0 = jnp.concatenate([jnp.eye(DH, dtype=f32),
                          jnp.zeros((DH, DH), f32)], axis=1)
    h1 = jnp.concatenate([jnp.zeros((DH, DH), f32),
                          jnp.eye(DH, dtype=f32)], axis=1)
    pa = jnp.kron(i4, h0)
    pb = jnp.kron(i4, h1)

    norm_tabs = []
    for k in range(L):
        wg = jnp.kron(i2, W_gc[k])
        wb = jnp.kron(i2, W_bi[k])
        b2 = jnp.tile(b_gc[k] + b_bi[k], (1, 2))
        side_sp = _spmm(ego_nm.reshape(2 * NPAD, DH), rc, valp)
        ego_nm, norm_k = _tc_transform(side_sp.reshape(NC, P4, 128), ego_nm,
                                       pa, pb, wg, wb, b2, gg)
        norm_tabs.append(norm_k)

    idx_all = jnp.concatenate(
        [users.astype(jnp.int32), items.astype(jnp.int32) + N_USER]
    ).reshape(32, 2, 128)
    out4 = _final_gather(ego0_pad, norm_tabs[0].reshape(NPAD, D),
                         norm_tabs[1].reshape(NPAD, D),
                         norm_tabs[2].reshape(NPAD, D), idx_all)
    res = out4.reshape(4, 2 * B, D).transpose(1, 0, 2).reshape(2 * B, 4 * D)
    return res[:B], res[B:]


# speedup vs baseline: 12.1167x; 1.0228x over previous
import functools

import jax
import jax.numpy as jnp
from jax import lax
from jax.experimental import pallas as pl
from jax.experimental.pallas import tpu as pltpu
from jax.experimental.pallas import tpu_sc as plsc

N_USER = 25000
N_ITEM = 25000
NNODE = N_USER + N_ITEM
D = 64
DH = 32
L = 3
E = 800000
B = 4096

NC, NS = 2, 16
K = 256
NSUB = K // 128
NBLK = 196
ET = NBLK * K
EP = NS * ET
NPAD = 50176
RPT = NPAD // NS
P4 = NPAD // 4
P2 = NPAD // 2
RP = 1792
GRID = P4 // RP

_mesh = plsc.VectorSubcoreMesh(core_axis_name="c", subcore_axis_name="s")
_sc_params = pltpu.CompilerParams(use_tc_tiling_on_sc=False)


@functools.partial(
    pl.kernel,
    out_type=jax.ShapeDtypeStruct((NC, NPAD, DH), jnp.float32),
    mesh=_mesh,
    scratch_types=[
        pltpu.MemorySpace.VMEM_SHARED((NPAD, DH), jnp.float32),
        pltpu.MemorySpace.VMEM((4, NSUB, 128), jnp.int32),
        pltpu.MemorySpace.VMEM((4, NSUB, 128), jnp.int32),
        pltpu.MemorySpace.VMEM((4, K), jnp.float32),
        pltpu.MemorySpace.VMEM((2, NSUB, 128), jnp.int32),
        pltpu.MemorySpace.VMEM((2, K, DH), jnp.float32),
        pltpu.SemaphoreType.DMA((4,)),
        pltpu.SemaphoreType.DMA((2,)),
        pltpu.SemaphoreType.DMA((2,)),
    ],
    compiler_params=_sc_params,
)
def _spmm(tbl_hbm, rc_hbm, val_hbm, out_hbm,
          acc, colb, rowb, valb, rowS, gath, isem, gsem, ssem):
    c = lax.axis_index("c")
    t = lax.axis_index("s")

    def zrow(i, _):
        gath[0, i, pl.ds(0, 16)] = jnp.zeros((16,), jnp.float32)
        gath[0, i, pl.ds(16, 16)] = jnp.zeros((16,), jnp.float32)
        return 0
    lax.fori_loop(0, K, zrow, 0)
    zd = []
    for off in range(0, RPT, K):
        sz = min(K, RPT - off)
        zd.append(pltpu.async_copy(gath.at[0, pl.ds(0, sz)],
                                   acc.at[pl.ds(t * RPT + off, sz)],
                                   ssem.at[0]))
    for dd in zd:
        dd.wait()

    off_c = c

    def issue_idx(b, r):
        pltpu.async_copy(rc_hbm.at[1, t, b], colb.at[r], isem.at[r])
        pltpu.async_copy(rc_hbm.at[0, t, b], rowb.at[r], isem.at[r])
        pltpu.async_copy(val_hbm.at[t, b], valb.at[r], isem.at[r])

    def drain_idx(b, r):
        pltpu.make_async_copy(rc_hbm.at[1, t, b], colb.at[r],
                              isem.at[r]).wait()
        pltpu.make_async_copy(rc_hbm.at[0, t, b], rowb.at[r],
                              isem.at[r]).wait()
        pltpu.make_async_copy(val_hbm.at[t, b], valb.at[r],
                              isem.at[r]).wait()

    def drain_scat(s):
        for j in range(NSUB):
            pltpu.make_async_copy(gath.at[s, pl.ds(j * 128, 128)],
                                  acc.at[rowS.at[s, j]], ssem.at[s]).wait()

    def produce(b, r, s, first):
        drain_idx(b, r)

        def obody(ii, _):
            for j in range(NSUB):
                colb[r, j, pl.ds(ii * 16, 16)] = (
                    colb[r, j, pl.ds(ii * 16, 16)] * 2 + off_c)
            return 0
        lax.fori_loop(0, 8, obody, 0, unroll=True)

        def after_credit():
            for j in range(NSUB):
                def cbody(ii, _):
                    rowS[s, j, pl.ds(ii * 16, 16)] = rowb[r, j,
                                                          pl.ds(ii * 16, 16)]
                    return 0
                lax.fori_loop(0, 8, cbody, 0, unroll=True)

        if first:
            @pl.when(b >= 2)
            def _():
                drain_scat(s)
        else:
            drain_scat(s)
        after_credit()
        return [pltpu.async_copy(tbl_hbm.at[colb.at[r, j]],
                                 gath.at[s, pl.ds(j * 128, 128)], gsem.at[s])
                for j in range(NSUB)]

    def scale(s, r):
        @plsc.parallel_loop(0, K // 16, 1, unroll=2)
        def _(g):
            vv = valb[r, pl.ds(g * 16, 16)]
            base = g * 16
            for j in range(16):
                v = vv[j]
                gath[s, base + j, pl.ds(0, 16)] = (
                    gath[s, base + j, pl.ds(0, 16)] * v)
                gath[s, base + j, pl.ds(16, 16)] = (
                    gath[s, base + j, pl.ds(16, 16)] * v)

    def consume(x, r, s):
        pltpu.make_async_copy(tbl_hbm.at[colb.at[r, 0]],
                              gath.at[s, pl.ds(0, 128)], gsem.at[s]).wait()
        pltpu.make_async_copy(tbl_hbm.at[colb.at[r, 1]],
                              gath.at[s, pl.ds(128, 128)], gsem.at[s]).wait()
        scale(s, r)
        for j in range(NSUB):
            pltpu.async_copy(gath.at[s, pl.ds(j * 128, 128)],
                             acc.at[rowS.at[s, j]], ssem.at[s], add=True)

    for bb in range(3):
        issue_idx(bb, bb)
    plsc.subcore_barrier()

    NB = ET // K

    def quad_body(o, _):
        for u in range(4):
            b = 4 * o + u
            s = u % 2
            gd = produce(b, u, s, first=(u < 2))
            xu = (u - 1) % 4
            xs = (u - 1) % 2
            pref = b + 2
            if u == 0:
                @pl.when(o > 0)
                def _():
                    consume(b - 1, xu, xs)
                    issue_idx(pref, (u + 2) % 4)
            elif u == 1:
                consume(b - 1, xu, xs)
                issue_idx(pref, (u + 2) % 4)
            else:
                consume(b - 1, xu, xs)

                @pl.when(o < NB // 4 - 1)
                def _():
                    issue_idx(pref, (u + 2) % 4)
        return 0

    lax.fori_loop(0, NB // 4, quad_body, 0)
    consume(NB - 1, (NB - 1) % 4, (NB - 1) % 2)
    drain_scat(0)
    drain_scat(1)
    plsc.subcore_barrier()
    pltpu.sync_copy(acc.at[pl.ds(t * RPT, RPT)],
                    out_hbm.at[c, pl.ds(t * RPT, RPT)])


def _tc_transform_body(s_ref, e_ref, pa_ref, pb_ref, wg_ref, wb_ref,
                       b2_ref, gg_ref, ego_o_ref, norm_o_ref):
    f32 = jnp.float32
    snm = (jnp.dot(s_ref[0], pa_ref[...], preferred_element_type=f32)
           + jnp.dot(s_ref[1], pb_ref[...], preferred_element_type=f32)
           ).reshape(2 * RP, 128)
    e = e_ref[...]
    z = (jnp.dot(snm, wg_ref[...], preferred_element_type=f32)
         + jnp.dot(e * snm, wb_ref[...], preferred_element_type=f32)
         + b2_ref[...])
    y = jnp.where(z >= 0, z, 0.2 * z)
    nrmb = jnp.dot(y * y, gg_ref[...], preferred_element_type=f32)
    norm_o_ref[...] = y / jnp.maximum(jnp.sqrt(nrmb), 1e-12)
    ego_o_ref[...] = y


_tc_transform = pl.pallas_call(
    _tc_transform_body,
    grid=(GRID,),
    in_specs=[
        pl.BlockSpec((NC, RP, 128), lambda i: (0, i, 0)),
        pl.BlockSpec((2 * RP, 128), lambda i: (i, 0)),
        pl.BlockSpec((128, 256), lambda i: (0, 0)),
        pl.BlockSpec((128, 256), lambda i: (0, 0)),
        pl.BlockSpec((128, 128), lambda i: (0, 0)),
        pl.BlockSpec((128, 128), lambda i: (0, 0)),
        pl.BlockSpec((1, 128), lambda i: (0, 0)),
        pl.BlockSpec((128, 128), lambda i: (0, 0)),
    ],
    out_specs=[
        pl.BlockSpec((2 * RP, 128), lambda i: (i, 0)),
        pl.BlockSpec((2 * RP, 128), lambda i: (i, 0)),
    ],
    out_shape=[
        jax.ShapeDtypeStruct((P2, 128), jnp.float32),
        jax.ShapeDtypeStruct((P2, 128), jnp.float32),
    ],
)


@functools.partial(
    pl.kernel,
    out_type=jax.ShapeDtypeStruct((4 * 2 * B, D), jnp.float32),
    mesh=_mesh,
    scratch_types=[
        pltpu.MemorySpace.VMEM((2, 128), jnp.int32),
        pltpu.MemorySpace.VMEM((4, 128, D), jnp.float32),
        pltpu.SemaphoreType.DMA,
    ],
    compiler_params=_sc_params,
)
def _final_gather(t0, t1, t2, t3, idx_hbm, out_hbm, idxb, gbuf, sem):
    c = lax.axis_index("c")
    t = lax.axis_index("s")
    wid = t * NC + c
    pltpu.sync_copy(idx_hbm.at[wid], idxb)
    nrows = 2 * B
    for j in range(2):
        gd = [pltpu.async_copy(tref.at[idxb.at[j]], gbuf.at[tab], sem)
              for tab, tref in enumerate((t0, t1, t2, t3))]
        for dd in gd:
            dd.wait()
        base = wid * 256 + j * 128
        for tab in range(4):
            pltpu.sync_copy(gbuf.at[tab],
                            out_hbm.at[pl.ds(tab * nrows + base, 128)])


def kernel(user_emb, item_emb, edge_values, W_gc, b_gc, W_bi, b_bi,
           edge_index, users, items):
    ego0 = jnp.concatenate([user_emb, item_emb], axis=0)
    ego0_pad = jnp.pad(ego0, ((0, NPAD - NNODE), (0, 0)))
    ego_nm = ego0_pad.reshape(P2, 128)

    rc = jnp.pad(edge_index.astype(jnp.int32),
                 ((0, 0), (0, EP - E))).reshape(2, NS, NBLK, NSUB, 128)
    valp = jnp.pad(edge_values, (0, EP - E)).reshape(NS, NBLK, K)

    f32 = jnp.float32
    i2 = jnp.eye(2, dtype=f32)
    i4 = jnp.eye(4, dtype=f32)
    gg = jnp.kron(i2, jnp.ones((D, D), f32))
    h0 = jnp.concatenate([jnp.eye(DH, dtype=f32),
                          jnp.zeros((DH, DH), f32)], axis=1)
    h1 = jnp.concatenate([jnp.zeros((DH, DH), f32),
                          jnp.eye(DH, dtype=f32)], axis=1)
    pa = jnp.kron(i4, h0)
    pb = jnp.kron(i4, h1)

    norm_tabs = []
    for k in range(L):
        wg = jnp.kron(i2, W_gc[k])
        wb = jnp.kron(i2, W_bi[k])
        b2 = jnp.tile(b_gc[k] + b_bi[k], (1, 2))
        side_sp = _spmm(ego_nm.reshape(2 * NPAD, DH), rc, valp)
        ego_nm, norm_k = _tc_transform(side_sp.reshape(NC, P4, 128), ego_nm,
                                       pa, pb, wg, wb, b2, gg)
        norm_tabs.append(norm_k)

    idx_all = jnp.concatenate(
        [users.astype(jnp.int32), items.astype(jnp.int32) + N_USER]
    ).reshape(32, 2, 128)
    out4 = _final_gather(ego0_pad, norm_tabs[0].reshape(NPAD, D),
                         norm_tabs[1].reshape(NPAD, D),
                         norm_tabs[2].reshape(NPAD, D), idx_all)
    res = out4.reshape(4, 2 * B, D).transpose(1, 0, 2).reshape(2 * B, 4 * D)
    return res[:B], res[B:]
